# Initial kernel scaffold; baseline (speedup 1.0000x reference)
import jax
import jax.numpy as jnp
from jax.experimental import pallas as pl


def kernel(xyz, atom_xyz, atomtypes, batch, atom_batch, W1, b1, W2, b2, W3, b3, watt, W4, b4, W5, b5, W6, b6):
    raise NotImplementedError("write your pallas kernel here")


# speedup vs baseline: 6.1862x; 6.1862x over previous
import jax
import jax.numpy as jnp
from jax.experimental import pallas as pl
from jax.experimental.pallas import tpu as pltpu

K = 16
AD = 6
CD = 16
BLK = 64
BIG = float("inf")


def _lrelu(x):
    return jnp.where(x >= 0, x, 0.2 * x)


def _prep_body(aty_ref, y0c_ref, y1c_ref, y2c_ref, W1_ref, b1_ref,
               W2_ref, b2_ref, W3_ref, b3_ref, T_ref):
    at = aty_ref[...]
    at = _lrelu(jnp.dot(at, W1_ref[...].T, preferred_element_type=jnp.float32, precision=jax.lax.Precision.HIGHEST)
                + b1_ref[...])
    at = _lrelu(jnp.dot(at, W2_ref[...].T, preferred_element_type=jnp.float32, precision=jax.lax.Precision.HIGHEST)
                + b2_ref[...])
    at = _lrelu(jnp.dot(at, W3_ref[...].T, preferred_element_type=jnp.float32, precision=jax.lax.Precision.HIGHEST)
                + b3_ref[...])
    T_ref[...] = jnp.concatenate(
        [at, y0c_ref[...] * at, y1c_ref[...] * at, y2c_ref[...] * at], axis=1)


def _main_body(x0_ref, x1_ref, x2_ref, y0_ref, y1_ref, y2_ref, T_ref,
               watt_ref, W4_ref, b4_ref, W5_ref, b5_ref, W6_ref, b6_ref,
               out_ref, d_s, W_s):
    x0 = x0_ref[...]
    x1 = x1_ref[...]
    x2 = x2_ref[...]
    d = ((x0 - y0_ref[...]) ** 2 + (x1 - y1_ref[...]) ** 2
         + (x2 - y2_ref[...]) ** 2)
    d_s[...] = d
    W_s[...] = jnp.zeros_like(d)
    for k in range(K):
        d = d_s[...]
        m = jnp.min(d, axis=1, keepdims=True)
        wk = watt_ref[0, k] / (m + 1e-8)
        eq = d == m
        W_s[...] = jnp.where(eq, wk, W_s[...])
        d_s[...] = jnp.where(eq, BIG, d)
    A = jnp.dot(W_s[...], T_ref[...], preferred_element_type=jnp.float32, precision=jax.lax.Precision.HIGHEST)
    A0 = A[:, 0:CD]
    fx2 = jnp.zeros_like(A0)
    for dd, xc in ((0, x0), (1, x1), (2, x2)):
        fxd = xc * A0 - A[:, CD * (dd + 1):CD * (dd + 2)]
        fx2 = fx2 + fxd * fxd
    fx = jnp.sqrt(fx2)
    h = _lrelu(jnp.dot(fx, W4_ref[...].T, preferred_element_type=jnp.float32, precision=jax.lax.Precision.HIGHEST)
               + b4_ref[...])
    h = _lrelu(jnp.dot(h, W5_ref[...].T, preferred_element_type=jnp.float32, precision=jax.lax.Precision.HIGHEST)
               + b5_ref[...])
    out_ref[...] = (jnp.dot(h, W6_ref[...].T,
                            preferred_element_type=jnp.float32, precision=jax.lax.Precision.HIGHEST) + b6_ref[...])


def kernel(xyz, atom_xyz, atomtypes, batch, atom_batch,
           W1, b1, W2, b2, W3, b3, watt, W4, b4, W5, b5, W6, b6):
    N = xyz.shape[0]
    M = atom_xyz.shape[0]
    Mp = ((M + 127) // 128) * 128
    Np = ((N + BLK - 1) // BLK) * BLK

    aty_p = jnp.pad(atomtypes[:, :AD], ((0, Mp - M), (0, 0)))
    axyz_p = jnp.pad(atom_xyz, ((0, Mp - M), (0, 0)), constant_values=1e6)
    y0c = axyz_p[:, 0:1]
    y1c = axyz_p[:, 1:2]
    y2c = axyz_p[:, 2:3]
    y0r, y1r, y2r = y0c.reshape(1, Mp), y1c.reshape(1, Mp), y2c.reshape(1, Mp)
    x_p = jnp.pad(xyz, ((0, Np - N), (0, 0)))
    x0c, x1c, x2c = x_p[:, 0:1], x_p[:, 1:2], x_p[:, 2:3]
    watt2 = watt.reshape(1, K)
    b1r, b2r, b3r = b1.reshape(1, CD), b2.reshape(1, CD), b3.reshape(1, CD)
    b4r, b5r, b6r = b4.reshape(1, CD), b5.reshape(1, CD), b6.reshape(1, CD)

    T = pl.pallas_call(
        _prep_body,
        out_shape=jax.ShapeDtypeStruct((Mp, 4 * CD), jnp.float32),
    )(aty_p, y0c, y1c, y2c, W1, b1r, W2, b2r, W3, b3r)

    grid = (Np // BLK,)
    full = lambda i: (0, 0)
    blkcol = lambda i: (i, 0)
    out = pl.pallas_call(
        _main_body,
        grid=grid,
        in_specs=[
            pl.BlockSpec((BLK, 1), blkcol),
            pl.BlockSpec((BLK, 1), blkcol),
            pl.BlockSpec((BLK, 1), blkcol),
            pl.BlockSpec((1, Mp), full),
            pl.BlockSpec((1, Mp), full),
            pl.BlockSpec((1, Mp), full),
            pl.BlockSpec((Mp, 4 * CD), full),
            pl.BlockSpec(memory_space=pltpu.SMEM),
            pl.BlockSpec((CD, CD), full),
            pl.BlockSpec((1, CD), full),
            pl.BlockSpec((CD, CD), full),
            pl.BlockSpec((1, CD), full),
            pl.BlockSpec((CD, CD), full),
            pl.BlockSpec((1, CD), full),
        ],
        out_specs=pl.BlockSpec((BLK, CD), blkcol),
        out_shape=jax.ShapeDtypeStruct((Np, CD), jnp.float32),
        scratch_shapes=[
            pltpu.VMEM((BLK, Mp), jnp.float32),
            pltpu.VMEM((BLK, Mp), jnp.float32),
        ],
    )(x0c, x1c, x2c, y0r, y1r, y2r, T, watt2, W4, b4r, W5, b5r, W6, b6r)
    return out[:N]
